# Initial kernel scaffold; baseline (speedup 1.0000x reference)
import functools

import jax
import jax.numpy as jnp
from jax import lax
from jax.experimental import pallas as pl
from jax.experimental.pallas import tpu as pltpu
from jax.experimental.pallas import tpu_sc as plsc

N = 10000
NP = 10240
E = 320000
NW = 32
CHUNK = 128
CPW = 80
EP = NW * CPW * CHUNK
ROWS_PER_TILE = NP // 16
BLK = 1024
GRID = NP // BLK


def _make_prop(D):
    mesh = plsc.VectorSubcoreMesh(core_axis_name="c", subcore_axis_name="s")

    @functools.partial(
        pl.kernel,
        out_type=jax.ShapeDtypeStruct((2, NP, D), jnp.float32),
        mesh=mesh,
        scratch_types=[
            pltpu.VMEM((CPW, CHUNK), jnp.int32),
            pltpu.VMEM((CPW, CHUNK), jnp.int32),
            pltpu.VMEM((CHUNK, D), jnp.float32),
            pltpu.VMEM_SHARED((NP, D), jnp.float32),
            pltpu.SemaphoreType.DMA,
        ],
    )
    def prop(g, srcw, dstw, zinit, out, src_v, dst_v, rows_v, acc, sem):
        c = lax.axis_index("c")
        s = lax.axis_index("s")
        wid = s * 2 + c
        r0 = s * ROWS_PER_TILE
        pltpu.sync_copy(zinit.at[pl.ds(r0, ROWS_PER_TILE)],
                        acc.at[pl.ds(r0, ROWS_PER_TILE)])
        pltpu.sync_copy(srcw.at[wid], src_v)
        pltpu.sync_copy(dstw.at[wid], dst_v)
        plsc.subcore_barrier()

        def body(j, carry):
            pltpu.async_copy(g.at[src_v.at[j]], rows_v, sem).wait()
            pltpu.sync_copy(rows_v, acc.at[dst_v.at[j]], add=True)
            return carry

        lax.fori_loop(0, CPW, body, 0)
        plsc.subcore_barrier()
        pltpu.sync_copy(acc.at[pl.ds(r0, ROWS_PER_TILE)],
                        out.at[c, pl.ds(r0, ROWS_PER_TILE)])

    return prop


_prop128 = _make_prop(128)
_prop16 = _make_prop(16)


def _tcA_body(deg2_ref, x_ref, w1_ref, dinv_ref, g1_ref):
    deg = deg2_ref[0, :, 0] + deg2_ref[1, :, 0] + 1.0
    dinv = lax.rsqrt(deg)
    dinv_ref[...] = dinv[:, None]
    h = jnp.dot(x_ref[...], w1_ref[...], preferred_element_type=jnp.float32)
    g1_ref[...] = h * dinv[:, None]


def _tcB_body(p_ref, g1_ref, dinv_ref, w2_ref, b1_ref, g2_ref):
    dinv = dinv_ref[...]
    tot = (g1_ref[...] + p_ref[0] + p_ref[1]) * dinv + b1_ref[...]
    z = jnp.maximum(tot, 0.0)
    h2 = jnp.dot(z, w2_ref[...], preferred_element_type=jnp.float32)
    g2_ref[...] = h2 * dinv


def _tcC_body(q_ref, g2_ref, dinv_ref, b2_ref, out_ref):
    out_ref[...] = ((g2_ref[...] + q_ref[0] + q_ref[1]) * dinv_ref[...]
                    + b2_ref[...])


def _tcA(deg2, xp, W1):
    return pl.pallas_call(
        _tcA_body,
        grid=(GRID,),
        in_specs=[
            pl.BlockSpec((2, BLK, 16), lambda i: (0, i, 0)),
            pl.BlockSpec((BLK, 128), lambda i: (i, 0)),
            pl.BlockSpec((128, 128), lambda i: (0, 0)),
        ],
        out_specs=[
            pl.BlockSpec((BLK, 1), lambda i: (i, 0)),
            pl.BlockSpec((BLK, 128), lambda i: (i, 0)),
        ],
        out_shape=[
            jax.ShapeDtypeStruct((NP, 1), jnp.float32),
            jax.ShapeDtypeStruct((NP, 128), jnp.float32),
        ],
    )(deg2, xp, W1)


def _tcB(p, g1, dinv, W2, b1):
    return pl.pallas_call(
        _tcB_body,
        grid=(GRID,),
        in_specs=[
            pl.BlockSpec((2, BLK, 128), lambda i: (0, i, 0)),
            pl.BlockSpec((BLK, 128), lambda i: (i, 0)),
            pl.BlockSpec((BLK, 1), lambda i: (i, 0)),
            pl.BlockSpec((128, 16), lambda i: (0, 0)),
            pl.BlockSpec((1, 128), lambda i: (0, 0)),
        ],
        out_specs=pl.BlockSpec((BLK, 16), lambda i: (i, 0)),
        out_shape=jax.ShapeDtypeStruct((NP, 16), jnp.float32),
    )(p, g1, dinv, W2, b1)


def _tcC(q, g2, dinv, b2):
    return pl.pallas_call(
        _tcC_body,
        grid=(GRID,),
        in_specs=[
            pl.BlockSpec((2, BLK, 16), lambda i: (0, i, 0)),
            pl.BlockSpec((BLK, 16), lambda i: (i, 0)),
            pl.BlockSpec((BLK, 1), lambda i: (i, 0)),
            pl.BlockSpec((1, 16), lambda i: (0, 0)),
        ],
        out_specs=pl.BlockSpec((BLK, 16), lambda i: (i, 0)),
        out_shape=jax.ShapeDtypeStruct((NP, 16), jnp.float32),
    )(q, g2, dinv, b2)


def kernel(x, edge_index, W1, b1, W2, b2):
    src = edge_index[0].astype(jnp.int32)
    dst = edge_index[1].astype(jnp.int32)
    pad = EP - E
    srcp = jnp.concatenate([src, jnp.full((pad,), N, jnp.int32)]).reshape(
        NW, CPW, CHUNK)
    dstp = jnp.concatenate([dst, jnp.full((pad,), N + 1, jnp.int32)]).reshape(
        NW, CPW, CHUNK)
    xp = jnp.zeros((NP, 128), jnp.float32).at[:N].set(x)
    ones16 = jnp.ones((NP, 16), jnp.float32)
    z16 = jnp.zeros((NP, 16), jnp.float32)
    z128 = jnp.zeros((NP, 128), jnp.float32)

    deg2 = _prop16(ones16, srcp, dstp, z16)
    dinv, g1 = _tcA(deg2, xp, W1)
    p = _prop128(g1, srcp, dstp, z128)
    g2 = _tcB(p, g1, dinv, W2, b1.reshape(1, 128))
    q = _prop16(g2, srcp, dstp, z16)
    outp = _tcC(q, g2, dinv, b2.reshape(1, 16))
    return outp[:N]


# speedup vs baseline: 11.8568x; 11.8568x over previous
import functools

import jax
import jax.numpy as jnp
from jax import lax
from jax.experimental import pallas as pl
from jax.experimental.pallas import tpu as pltpu
from jax.experimental.pallas import tpu_sc as plsc

N = 10000
NP = 10240
E = 320000
NW = 32
CHUNK = 128
CPW = 80
EP = NW * CPW * CHUNK
ROWS_PER_TILE = NP // 16
BLK = 1024
GRID = NP // BLK


def _make_prop(D):
    mesh = plsc.VectorSubcoreMesh(core_axis_name="c", subcore_axis_name="s")

    @functools.partial(
        pl.kernel,
        out_type=jax.ShapeDtypeStruct((2, NP, D), jnp.float32),
        mesh=mesh,
        compiler_params=pltpu.CompilerParams(use_tc_tiling_on_sc=(D == 128)),
        scratch_types=[
            pltpu.VMEM((CPW, CHUNK), jnp.int32),
            pltpu.VMEM((CPW, CHUNK), jnp.int32),
            pltpu.VMEM((CHUNK, D), jnp.float32),
            pltpu.VMEM_SHARED((NP, D), jnp.float32),
            pltpu.SemaphoreType.DMA,
        ],
    )
    def prop(g, srcw, dstw, zinit, out, src_v, dst_v, rows_v, acc, sem):
        c = lax.axis_index("c")
        s = lax.axis_index("s")
        wid = s * 2 + c
        r0 = s * ROWS_PER_TILE
        pltpu.sync_copy(zinit.at[pl.ds(r0, ROWS_PER_TILE)],
                        acc.at[pl.ds(r0, ROWS_PER_TILE)])
        pltpu.sync_copy(srcw.at[wid], src_v)
        pltpu.sync_copy(dstw.at[wid], dst_v)
        plsc.subcore_barrier()

        def body(j, carry):
            pltpu.async_copy(g.at[src_v.at[j]], rows_v, sem).wait()
            pltpu.sync_copy(rows_v, acc.at[dst_v.at[j]], add=True)
            return carry

        lax.fori_loop(0, CPW, body, 0)
        plsc.subcore_barrier()
        pltpu.sync_copy(acc.at[pl.ds(r0, ROWS_PER_TILE)],
                        out.at[c, pl.ds(r0, ROWS_PER_TILE)])

    return prop


_prop128 = _make_prop(128)
_prop16 = _make_prop(16)


def _tcA_body(deg2_ref, x_ref, w1_ref, dinv_ref, g1_ref):
    deg = deg2_ref[0, :, 0] + deg2_ref[1, :, 0] + 1.0
    dinv = lax.rsqrt(deg)
    dinv_ref[...] = dinv[:, None]
    h = jnp.dot(x_ref[...], w1_ref[...], preferred_element_type=jnp.float32)
    g1_ref[...] = h * dinv[:, None]


def _tcB_body(p_ref, g1_ref, dinv_ref, w2_ref, b1_ref, g2_ref):
    dinv = dinv_ref[...]
    tot = (g1_ref[...] + p_ref[0] + p_ref[1]) * dinv + b1_ref[...]
    z = jnp.maximum(tot, 0.0)
    h2 = jnp.dot(z, w2_ref[...], preferred_element_type=jnp.float32)
    g2_ref[...] = h2 * dinv


def _tcC_body(q_ref, g2_ref, dinv_ref, b2_ref, out_ref):
    out_ref[...] = ((g2_ref[...] + q_ref[0] + q_ref[1]) * dinv_ref[...]
                    + b2_ref[...])


def _tcA(deg2, xp, W1):
    return pl.pallas_call(
        _tcA_body,
        grid=(GRID,),
        in_specs=[
            pl.BlockSpec((2, BLK, 16), lambda i: (0, i, 0)),
            pl.BlockSpec((BLK, 128), lambda i: (i, 0)),
            pl.BlockSpec((128, 128), lambda i: (0, 0)),
        ],
        out_specs=[
            pl.BlockSpec((BLK, 1), lambda i: (i, 0)),
            pl.BlockSpec((BLK, 128), lambda i: (i, 0)),
        ],
        out_shape=[
            jax.ShapeDtypeStruct((NP, 1), jnp.float32),
            jax.ShapeDtypeStruct((NP, 128), jnp.float32),
        ],
    )(deg2, xp, W1)


def _tcB(p, g1, dinv, W2, b1):
    return pl.pallas_call(
        _tcB_body,
        grid=(GRID,),
        in_specs=[
            pl.BlockSpec((2, BLK, 128), lambda i: (0, i, 0)),
            pl.BlockSpec((BLK, 128), lambda i: (i, 0)),
            pl.BlockSpec((BLK, 1), lambda i: (i, 0)),
            pl.BlockSpec((128, 16), lambda i: (0, 0)),
            pl.BlockSpec((1, 128), lambda i: (0, 0)),
        ],
        out_specs=pl.BlockSpec((BLK, 16), lambda i: (i, 0)),
        out_shape=jax.ShapeDtypeStruct((NP, 16), jnp.float32),
    )(p, g1, dinv, W2, b1)


def _tcC(q, g2, dinv, b2):
    return pl.pallas_call(
        _tcC_body,
        grid=(GRID,),
        in_specs=[
            pl.BlockSpec((2, BLK, 16), lambda i: (0, i, 0)),
            pl.BlockSpec((BLK, 16), lambda i: (i, 0)),
            pl.BlockSpec((BLK, 1), lambda i: (i, 0)),
            pl.BlockSpec((1, 16), lambda i: (0, 0)),
        ],
        out_specs=pl.BlockSpec((BLK, 16), lambda i: (i, 0)),
        out_shape=jax.ShapeDtypeStruct((NP, 16), jnp.float32),
    )(q, g2, dinv, b2)


def kernel(x, edge_index, W1, b1, W2, b2):
    src = edge_index[0].astype(jnp.int32)
    dst = edge_index[1].astype(jnp.int32)
    pad = EP - E
    srcp = jnp.concatenate([src, jnp.full((pad,), N, jnp.int32)]).reshape(
        NW, CPW, CHUNK)
    dstp = jnp.concatenate([dst, jnp.full((pad,), N + 1, jnp.int32)]).reshape(
        NW, CPW, CHUNK)
    xp = jnp.zeros((NP, 128), jnp.float32).at[:N].set(x)
    ones16 = jnp.ones((NP, 16), jnp.float32)
    z16 = jnp.zeros((NP, 16), jnp.float32)
    z128 = jnp.zeros((NP, 128), jnp.float32)

    deg2 = _prop16(ones16, srcp, dstp, z16)
    dinv, g1 = _tcA(deg2, xp, W1)
    p = _prop128(g1, srcp, dstp, z128)
    g2 = _tcB(p, g1, dinv, W2, b1.reshape(1, 128))
    q = _prop16(g2, srcp, dstp, z16)
    outp = _tcC(q, g2, dinv, b2.reshape(1, 16))
    return outp[:N]


# speedup vs baseline: 13.2969x; 1.1215x over previous
import functools

import jax
import jax.numpy as jnp
from jax import lax
from jax.experimental import pallas as pl
from jax.experimental.pallas import tpu as pltpu
from jax.experimental.pallas import tpu_sc as plsc

N = 10000
NP = 10240
E = 320000
NW = 32
CHUNK = 128
CPW = 80
EP = NW * CPW * CHUNK
ROWS_PER_TILE = NP // 16
BLK = 1024
GRID = NP // BLK


WAVE = 16
WAVES = CPW // WAVE


def _make_prop(D):
    nbuf = 2 if D == 128 else 4
    mesh = plsc.VectorSubcoreMesh(core_axis_name="c", subcore_axis_name="s")

    @functools.partial(
        pl.kernel,
        out_type=jax.ShapeDtypeStruct((2, NP, D), jnp.float32),
        mesh=mesh,
        compiler_params=pltpu.CompilerParams(use_tc_tiling_on_sc=False),
        scratch_types=[
            pltpu.VMEM((WAVE, CHUNK), jnp.int32),
            pltpu.VMEM((WAVE, CHUNK), jnp.int32),
            [pltpu.VMEM((CHUNK, D), jnp.float32)] * nbuf,
            pltpu.VMEM_SHARED((NP, D), jnp.float32),
            [pltpu.SemaphoreType.DMA] * nbuf,
        ],
    )
    def prop(g, srcw, dstw, zinit, out, src_v, dst_v, rows_v, acc, sems):
        c = lax.axis_index("c")
        s = lax.axis_index("s")
        wid = s * 2 + c
        r0 = s * ROWS_PER_TILE
        pltpu.sync_copy(zinit.at[pl.ds(r0, ROWS_PER_TILE)],
                        acc.at[pl.ds(r0, ROWS_PER_TILE)])
        plsc.subcore_barrier()

        def wave_body(w, carry):
            pltpu.sync_copy(srcw.at[wid, w], src_v)
            pltpu.sync_copy(dstw.at[wid, w], dst_v)
            for b in range(nbuf):
                pltpu.async_copy(g.at[src_v.at[b]], rows_v[b], sems[b])

            def body(jj, carry2):
                for b in range(nbuf):
                    j = jj * nbuf + b
                    pltpu.make_async_copy(
                        g.at[src_v.at[j]], rows_v[b], sems[b]).wait()
                    pltpu.sync_copy(rows_v[b], acc.at[dst_v.at[j]], add=True)

                    @pl.when(j + nbuf < WAVE)
                    def _():
                        pltpu.async_copy(
                            g.at[src_v.at[j + nbuf]], rows_v[b], sems[b])
                return carry2

            lax.fori_loop(0, WAVE // nbuf, body, 0)
            return carry

        lax.fori_loop(0, WAVES, wave_body, 0)
        plsc.subcore_barrier()
        pltpu.sync_copy(acc.at[pl.ds(r0, ROWS_PER_TILE)],
                        out.at[c, pl.ds(r0, ROWS_PER_TILE)])

    return prop


_prop128 = _make_prop(128)
_prop16 = _make_prop(16)


def _make_deg():
    mesh = plsc.VectorSubcoreMesh(core_axis_name="c", subcore_axis_name="s")

    @functools.partial(
        pl.kernel,
        out_type=jax.ShapeDtypeStruct((2, NP, 16), jnp.float32),
        mesh=mesh,
        compiler_params=pltpu.CompilerParams(use_tc_tiling_on_sc=False),
        scratch_types=[
            pltpu.VMEM((WAVES, WAVE, CHUNK), jnp.int32),
            pltpu.VMEM((CHUNK, 16), jnp.float32),
            pltpu.VMEM_SHARED((NP, 16), jnp.float32),
            pltpu.SemaphoreType.DMA,
        ],
    )
    def deg(ones_hbm, dstw, zinit, out, dst_v, ones_v, acc, sem):
        c = lax.axis_index("c")
        s = lax.axis_index("s")
        wid = s * 2 + c
        r0 = s * ROWS_PER_TILE
        pltpu.sync_copy(zinit.at[pl.ds(r0, ROWS_PER_TILE)],
                        acc.at[pl.ds(r0, ROWS_PER_TILE)])
        pltpu.sync_copy(dstw.at[wid], dst_v)
        pltpu.sync_copy(ones_hbm, ones_v)
        plsc.subcore_barrier()

        def fire(j, carry):
            pltpu.async_copy(
                ones_v, acc.at[dst_v.at[j // WAVE, j % WAVE]], sem, add=True)
            return carry

        lax.fori_loop(0, CPW, fire, 0)

        def drain(j, carry):
            pltpu.make_async_copy(
                ones_v, acc.at[dst_v.at[j // WAVE, j % WAVE]], sem).wait()
            return carry

        lax.fori_loop(0, CPW, drain, 0)
        plsc.subcore_barrier()
        pltpu.sync_copy(acc.at[pl.ds(r0, ROWS_PER_TILE)],
                        out.at[c, pl.ds(r0, ROWS_PER_TILE)])

    return deg


_deg16 = _make_deg()


def _tcA_body(deg2_ref, x_ref, w1_ref, dinv_ref, g1_ref):
    deg = deg2_ref[0, :, 0] + deg2_ref[1, :, 0] + 1.0
    dinv = lax.rsqrt(deg)
    dinv_ref[...] = dinv[:, None]
    h = jnp.dot(x_ref[...], w1_ref[...], preferred_element_type=jnp.float32)
    g1_ref[...] = h * dinv[:, None]


def _tcB_body(p_ref, g1_ref, dinv_ref, w2_ref, b1_ref, g2_ref):
    dinv = dinv_ref[...]
    tot = (g1_ref[...] + p_ref[0] + p_ref[1]) * dinv + b1_ref[...]
    z = jnp.maximum(tot, 0.0)
    h2 = jnp.dot(z, w2_ref[...], preferred_element_type=jnp.float32)
    g2_ref[...] = h2 * dinv


def _tcC_body(q_ref, g2_ref, dinv_ref, b2_ref, out_ref):
    out_ref[...] = ((g2_ref[...] + q_ref[0] + q_ref[1]) * dinv_ref[...]
                    + b2_ref[...])


def _tcA(deg2, xp, W1):
    return pl.pallas_call(
        _tcA_body,
        grid=(GRID,),
        in_specs=[
            pl.BlockSpec((2, BLK, 16), lambda i: (0, i, 0)),
            pl.BlockSpec((BLK, 128), lambda i: (i, 0)),
            pl.BlockSpec((128, 128), lambda i: (0, 0)),
        ],
        out_specs=[
            pl.BlockSpec((BLK, 1), lambda i: (i, 0)),
            pl.BlockSpec((BLK, 128), lambda i: (i, 0)),
        ],
        out_shape=[
            jax.ShapeDtypeStruct((NP, 1), jnp.float32),
            jax.ShapeDtypeStruct((NP, 128), jnp.float32),
        ],
    )(deg2, xp, W1)


def _tcB(p, g1, dinv, W2, b1):
    return pl.pallas_call(
        _tcB_body,
        grid=(GRID,),
        in_specs=[
            pl.BlockSpec((2, BLK, 128), lambda i: (0, i, 0)),
            pl.BlockSpec((BLK, 128), lambda i: (i, 0)),
            pl.BlockSpec((BLK, 1), lambda i: (i, 0)),
            pl.BlockSpec((128, 16), lambda i: (0, 0)),
            pl.BlockSpec((1, 128), lambda i: (0, 0)),
        ],
        out_specs=pl.BlockSpec((BLK, 16), lambda i: (i, 0)),
        out_shape=jax.ShapeDtypeStruct((NP, 16), jnp.float32),
    )(p, g1, dinv, W2, b1)


def _tcC(q, g2, dinv, b2):
    return pl.pallas_call(
        _tcC_body,
        grid=(GRID,),
        in_specs=[
            pl.BlockSpec((2, BLK, 16), lambda i: (0, i, 0)),
            pl.BlockSpec((BLK, 16), lambda i: (i, 0)),
            pl.BlockSpec((BLK, 1), lambda i: (i, 0)),
            pl.BlockSpec((1, 16), lambda i: (0, 0)),
        ],
        out_specs=pl.BlockSpec((BLK, 16), lambda i: (i, 0)),
        out_shape=jax.ShapeDtypeStruct((NP, 16), jnp.float32),
    )(q, g2, dinv, b2)


def kernel(x, edge_index, W1, b1, W2, b2):
    src = edge_index[0].astype(jnp.int32)
    dst = edge_index[1].astype(jnp.int32)
    pad = EP - E
    srcp = jnp.concatenate([src, jnp.full((pad,), N, jnp.int32)]).reshape(
        NW, WAVES, WAVE, CHUNK)
    dstp = jnp.concatenate([dst, jnp.full((pad,), N + 1, jnp.int32)]).reshape(
        NW, WAVES, WAVE, CHUNK)
    xp = jnp.zeros((NP, 128), jnp.float32).at[:N].set(x)
    ones_tile = jnp.ones((CHUNK, 16), jnp.float32)
    z16 = jnp.zeros((NP, 16), jnp.float32)
    z128 = jnp.zeros((NP, 128), jnp.float32)

    deg2 = _deg16(ones_tile, dstp, z16)
    dinv, g1 = _tcA(deg2, xp, W1)
    p = _prop128(g1, srcp, dstp, z128)
    g2 = _tcB(p, g1, dinv, W2, b1.reshape(1, 128))
    q = _prop16(g2, srcp, dstp, z16)
    outp = _tcC(q, g2, dinv, b2.reshape(1, 16))
    return outp[:N]


# speedup vs baseline: 13.4301x; 1.0100x over previous
import functools

import jax
import jax.numpy as jnp
from jax import lax
from jax.experimental import pallas as pl
from jax.experimental.pallas import tpu as pltpu
from jax.experimental.pallas import tpu_sc as plsc

N = 10000
NP = 10240
E = 320000
NW = 32
CHUNK = 64
CPW = 160
EP = NW * CPW * CHUNK
ROWS_PER_TILE = NP // 16
BLK = 1024
GRID = NP // BLK


WPC = 80
WAVES = CPW // WPC
RING = 4


def _make_prop(D):
    mesh = plsc.VectorSubcoreMesh(core_axis_name="c", subcore_axis_name="s")

    @functools.partial(
        pl.kernel,
        out_type=jax.ShapeDtypeStruct((2, NP, D), jnp.float32),
        mesh=mesh,
        compiler_params=pltpu.CompilerParams(use_tc_tiling_on_sc=False),
        scratch_types=[
            pltpu.VMEM((WPC, CHUNK), jnp.int32),
            pltpu.VMEM((WPC, CHUNK), jnp.int32),
            [pltpu.VMEM((CHUNK, D), jnp.float32)] * RING,
            pltpu.VMEM_SHARED((NP, D), jnp.float32),
            [pltpu.SemaphoreType.DMA] * RING,
            [pltpu.SemaphoreType.DMA] * RING,
        ],
    )
    def prop(g, srcw, dstw, zinit, out, src_v, dst_v, rows_v, acc, gsem, ssem):
        c = lax.axis_index("c")
        s = lax.axis_index("s")
        wid = s * 2 + c
        r0 = s * ROWS_PER_TILE
        pltpu.sync_copy(zinit.at[pl.ds(r0, ROWS_PER_TILE)],
                        acc.at[pl.ds(r0, ROWS_PER_TILE)])
        plsc.subcore_barrier()

        def wave_body(w, carry):
            pltpu.sync_copy(srcw.at[wid, w], src_v)
            pltpu.sync_copy(dstw.at[wid, w], dst_v)
            for b in range(RING - 1):
                pltpu.async_copy(g.at[src_v.at[b]], rows_v[b], gsem[b])

            def body(jj, carry2):
                for b in range(RING):
                    j = jj * RING + b
                    pltpu.make_async_copy(
                        g.at[src_v.at[j]], rows_v[b], gsem[b]).wait()
                    pltpu.async_copy(
                        rows_v[b], acc.at[dst_v.at[j]], ssem[b], add=True)

                    bn = (b + RING - 1) % RING

                    @pl.when(j + RING - 1 < WPC)
                    def _():
                        @pl.when(j >= 1)
                        def _():
                            pltpu.make_async_copy(
                                rows_v[bn], acc.at[dst_v.at[j]],
                                ssem[bn]).wait()
                        pltpu.async_copy(
                            g.at[src_v.at[j + RING - 1]], rows_v[bn],
                            gsem[bn])
                return carry2

            lax.fori_loop(0, WPC // RING, body, 0)
            for b in range(RING):
                pltpu.make_async_copy(
                    rows_v[b], acc.at[dst_v.at[b]], ssem[b]).wait()
            return carry

        lax.fori_loop(0, WAVES, wave_body, 0)
        plsc.subcore_barrier()
        pltpu.sync_copy(acc.at[pl.ds(r0, ROWS_PER_TILE)],
                        out.at[c, pl.ds(r0, ROWS_PER_TILE)])

    return prop


_prop128 = _make_prop(128)
_prop16 = _make_prop(16)


def _make_deg():
    mesh = plsc.VectorSubcoreMesh(core_axis_name="c", subcore_axis_name="s")

    @functools.partial(
        pl.kernel,
        out_type=jax.ShapeDtypeStruct((2, NP, 16), jnp.float32),
        mesh=mesh,
        compiler_params=pltpu.CompilerParams(use_tc_tiling_on_sc=False),
        scratch_types=[
            pltpu.VMEM((WAVES, WPC, CHUNK), jnp.int32),
            pltpu.VMEM((CHUNK, 16), jnp.float32),
            pltpu.VMEM_SHARED((NP, 16), jnp.float32),
            pltpu.SemaphoreType.DMA,
        ],
    )
    def deg(ones_hbm, dstw, zinit, out, dst_v, ones_v, acc, sem):
        c = lax.axis_index("c")
        s = lax.axis_index("s")
        wid = s * 2 + c
        r0 = s * ROWS_PER_TILE
        pltpu.sync_copy(zinit.at[pl.ds(r0, ROWS_PER_TILE)],
                        acc.at[pl.ds(r0, ROWS_PER_TILE)])
        pltpu.sync_copy(dstw.at[wid], dst_v)
        pltpu.sync_copy(ones_hbm, ones_v)
        plsc.subcore_barrier()

        def fire(j, carry):
            pltpu.async_copy(
                ones_v, acc.at[dst_v.at[j // WPC, j % WPC]], sem, add=True)
            return carry

        lax.fori_loop(0, CPW, fire, 0)

        def drain(j, carry):
            pltpu.make_async_copy(
                ones_v, acc.at[dst_v.at[j // WPC, j % WPC]], sem).wait()
            return carry

        lax.fori_loop(0, CPW, drain, 0)
        plsc.subcore_barrier()
        pltpu.sync_copy(acc.at[pl.ds(r0, ROWS_PER_TILE)],
                        out.at[c, pl.ds(r0, ROWS_PER_TILE)])

    return deg


_deg16 = _make_deg()


def _tcA_body(deg2_ref, x_ref, w1_ref, dinv_ref, g1_ref):
    deg = deg2_ref[0, :, 0] + deg2_ref[1, :, 0] + 1.0
    dinv = lax.rsqrt(deg)
    dinv_ref[...] = dinv[:, None]
    h = jnp.dot(x_ref[...], w1_ref[...], preferred_element_type=jnp.float32)
    g1_ref[...] = h * dinv[:, None]


def _tcB_body(p_ref, g1_ref, dinv_ref, w2_ref, b1_ref, g2_ref):
    dinv = dinv_ref[...]
    tot = (g1_ref[...] + p_ref[0] + p_ref[1]) * dinv + b1_ref[...]
    z = jnp.maximum(tot, 0.0)
    h2 = jnp.dot(z, w2_ref[...], preferred_element_type=jnp.float32)
    g2_ref[...] = h2 * dinv


def _tcC_body(q_ref, g2_ref, dinv_ref, b2_ref, out_ref):
    out_ref[...] = ((g2_ref[...] + q_ref[0] + q_ref[1]) * dinv_ref[...]
                    + b2_ref[...])


def _tcA(deg2, xp, W1):
    return pl.pallas_call(
        _tcA_body,
        grid=(GRID,),
        in_specs=[
            pl.BlockSpec((2, BLK, 16), lambda i: (0, i, 0)),
            pl.BlockSpec((BLK, 128), lambda i: (i, 0)),
            pl.BlockSpec((128, 128), lambda i: (0, 0)),
        ],
        out_specs=[
            pl.BlockSpec((BLK, 1), lambda i: (i, 0)),
            pl.BlockSpec((BLK, 128), lambda i: (i, 0)),
        ],
        out_shape=[
            jax.ShapeDtypeStruct((NP, 1), jnp.float32),
            jax.ShapeDtypeStruct((NP, 128), jnp.float32),
        ],
    )(deg2, xp, W1)


def _tcB(p, g1, dinv, W2, b1):
    return pl.pallas_call(
        _tcB_body,
        grid=(GRID,),
        in_specs=[
            pl.BlockSpec((2, BLK, 128), lambda i: (0, i, 0)),
            pl.BlockSpec((BLK, 128), lambda i: (i, 0)),
            pl.BlockSpec((BLK, 1), lambda i: (i, 0)),
            pl.BlockSpec((128, 16), lambda i: (0, 0)),
            pl.BlockSpec((1, 128), lambda i: (0, 0)),
        ],
        out_specs=pl.BlockSpec((BLK, 16), lambda i: (i, 0)),
        out_shape=jax.ShapeDtypeStruct((NP, 16), jnp.float32),
    )(p, g1, dinv, W2, b1)


def _tcC(q, g2, dinv, b2):
    return pl.pallas_call(
        _tcC_body,
        grid=(GRID,),
        in_specs=[
            pl.BlockSpec((2, BLK, 16), lambda i: (0, i, 0)),
            pl.BlockSpec((BLK, 16), lambda i: (i, 0)),
            pl.BlockSpec((BLK, 1), lambda i: (i, 0)),
            pl.BlockSpec((1, 16), lambda i: (0, 0)),
        ],
        out_specs=pl.BlockSpec((BLK, 16), lambda i: (i, 0)),
        out_shape=jax.ShapeDtypeStruct((NP, 16), jnp.float32),
    )(q, g2, dinv, b2)


def kernel(x, edge_index, W1, b1, W2, b2):
    src = edge_index[0].astype(jnp.int32)
    dst = edge_index[1].astype(jnp.int32)
    pad = EP - E
    srcp = jnp.concatenate([src, jnp.full((pad,), N, jnp.int32)]).reshape(
        NW, WAVES, WPC, CHUNK)
    dstp = jnp.concatenate([dst, jnp.full((pad,), N + 1, jnp.int32)]).reshape(
        NW, WAVES, WPC, CHUNK)
    xp = jnp.zeros((NP, 128), jnp.float32).at[:N].set(x)
    ones_tile = jnp.ones((CHUNK, 16), jnp.float32)
    z16 = jnp.zeros((NP, 16), jnp.float32)
    z128 = jnp.zeros((NP, 128), jnp.float32)

    deg2 = _deg16(ones_tile, dstp, z16)
    dinv, g1 = _tcA(deg2, xp, W1)
    p = _prop128(g1, srcp, dstp, z128)
    g2 = _tcB(p, g1, dinv, W2, b1.reshape(1, 128))
    q = _prop16(g2, srcp, dstp, z16)
    outp = _tcC(q, g2, dinv, b2.reshape(1, 16))
    return outp[:N]


# speedup vs baseline: 32.0844x; 2.3890x over previous
import functools

import jax
import jax.numpy as jnp
from jax import lax
from jax.experimental import pallas as pl
from jax.experimental.pallas import tpu as pltpu
from jax.experimental.pallas import tpu_sc as plsc

N = 10000
NP = 10240
E = 320000
NW = 32
CHUNK = 64
CPW = 160
EP = NW * CPW * CHUNK
ROWS_PER_TILE = NP // 16
RING = 4
BLK = 1024
GRID = NP // BLK


def _make_prop(TW, NPASS):
    mesh = plsc.VectorSubcoreMesh(core_axis_name="c", subcore_axis_name="s")

    @functools.partial(
        pl.kernel,
        out_type=jax.ShapeDtypeStruct((NPASS, 2, NP, TW), jnp.float32),
        mesh=mesh,
        compiler_params=pltpu.CompilerParams(use_tc_tiling_on_sc=False),
        scratch_types=[
            pltpu.VMEM((CPW, CHUNK), jnp.int32),
            pltpu.VMEM((CPW, CHUNK), jnp.int32),
            [pltpu.VMEM((CHUNK, TW), jnp.float32)] * RING,
            pltpu.VMEM_SHARED((NP, TW), jnp.float32),
            pltpu.VMEM_SHARED((NP, TW), jnp.float32),
            [pltpu.SemaphoreType.DMA] * RING,
            [pltpu.SemaphoreType.DMA] * RING,
        ],
    )
    def prop(gs, srcw, dstw, zinit, out,
             src_v, dst_v, rows_v, gtab, acc, gsem, ssem):
        c = lax.axis_index("c")
        s = lax.axis_index("s")
        wid = s * 2 + c
        r0 = s * ROWS_PER_TILE
        pltpu.sync_copy(srcw.at[wid], src_v)
        pltpu.sync_copy(dstw.at[wid], dst_v)

        for p in range(NPASS):
            pltpu.sync_copy(gs[p].at[pl.ds(r0, ROWS_PER_TILE)],
                            gtab.at[pl.ds(r0, ROWS_PER_TILE)])
            pltpu.sync_copy(zinit.at[pl.ds(r0, ROWS_PER_TILE)],
                            acc.at[pl.ds(r0, ROWS_PER_TILE)])
            plsc.subcore_barrier()

            for b in range(RING - 1):
                pltpu.async_copy(gtab.at[src_v.at[b]], rows_v[b], gsem[b])

            def body(jj, carry):
                for b in range(RING):
                    j = jj * RING + b
                    pltpu.make_async_copy(
                        gtab.at[src_v.at[j]], rows_v[b], gsem[b]).wait()
                    pltpu.async_copy(
                        rows_v[b], acc.at[dst_v.at[j]], ssem[b], add=True)

                    bn = (b + RING - 1) % RING

                    @pl.when(j + RING - 1 < CPW)
                    def _():
                        @pl.when(j >= 1)
                        def _():
                            pltpu.make_async_copy(
                                rows_v[bn], acc.at[dst_v.at[j]],
                                ssem[bn]).wait()
                        pltpu.async_copy(
                            gtab.at[src_v.at[j + RING - 1]], rows_v[bn],
                            gsem[bn])
                return carry

            lax.fori_loop(0, CPW // RING, body, 0)
            for b in range(RING):
                pltpu.make_async_copy(
                    rows_v[b], acc.at[dst_v.at[b]], ssem[b]).wait()
            plsc.subcore_barrier()
            pltpu.sync_copy(acc.at[pl.ds(r0, ROWS_PER_TILE)],
                            out.at[p, c, pl.ds(r0, ROWS_PER_TILE)])

    return prop


_prop64x2 = _make_prop(64, 2)
_prop16x1 = _make_prop(16, 1)


def _make_deg():
    mesh = plsc.VectorSubcoreMesh(core_axis_name="c", subcore_axis_name="s")

    @functools.partial(
        pl.kernel,
        out_type=jax.ShapeDtypeStruct((2, NP, 16), jnp.float32),
        mesh=mesh,
        compiler_params=pltpu.CompilerParams(use_tc_tiling_on_sc=False),
        scratch_types=[
            pltpu.VMEM((CPW, CHUNK), jnp.int32),
            pltpu.VMEM((CHUNK, 16), jnp.float32),
            pltpu.VMEM_SHARED((NP, 16), jnp.float32),
            pltpu.SemaphoreType.DMA,
        ],
    )
    def deg(ones_hbm, dstw, zinit, out, dst_v, ones_v, acc, sem):
        c = lax.axis_index("c")
        s = lax.axis_index("s")
        wid = s * 2 + c
        r0 = s * ROWS_PER_TILE
        pltpu.sync_copy(zinit.at[pl.ds(r0, ROWS_PER_TILE)],
                        acc.at[pl.ds(r0, ROWS_PER_TILE)])
        pltpu.sync_copy(dstw.at[wid], dst_v)
        pltpu.sync_copy(ones_hbm, ones_v)
        plsc.subcore_barrier()

        def fire(j, carry):
            pltpu.async_copy(ones_v, acc.at[dst_v.at[j]], sem, add=True)
            return carry

        lax.fori_loop(0, CPW, fire, 0)

        def drain(j, carry):
            pltpu.make_async_copy(ones_v, acc.at[dst_v.at[j]], sem).wait()
            return carry

        lax.fori_loop(0, CPW, drain, 0)
        plsc.subcore_barrier()
        pltpu.sync_copy(acc.at[pl.ds(r0, ROWS_PER_TILE)],
                        out.at[c, pl.ds(r0, ROWS_PER_TILE)])

    return deg


_deg16 = _make_deg()


def _tcA_body(deg2_ref, x_ref, w1_ref, dinv_ref, ga_ref, gb_ref):
    deg = deg2_ref[0, :, 0] + deg2_ref[1, :, 0] + 1.0
    dinv = lax.rsqrt(deg)
    dinv_ref[...] = dinv[:, None]
    h = jnp.dot(x_ref[...], w1_ref[...], preferred_element_type=jnp.float32)
    g1 = h * dinv[:, None]
    ga_ref[...] = g1[:, :64]
    gb_ref[...] = g1[:, 64:]


def _tcB_body(p_ref, ga_ref, gb_ref, dinv_ref, w2_ref, b1_ref, g2_ref):
    dinv = dinv_ref[...]
    tot = jnp.concatenate(
        [ga_ref[...] + p_ref[0, 0] + p_ref[0, 1],
         gb_ref[...] + p_ref[1, 0] + p_ref[1, 1]], axis=1)
    z = jnp.maximum(tot * dinv + b1_ref[...], 0.0)
    h2 = jnp.dot(z, w2_ref[...], preferred_element_type=jnp.float32)
    g2_ref[...] = h2 * dinv


def _tcC_body(q_ref, g2_ref, dinv_ref, b2_ref, out_ref):
    out_ref[...] = ((g2_ref[...] + q_ref[0, 0] + q_ref[0, 1])
                    * dinv_ref[...] + b2_ref[...])


def _tcA(deg2, xp, W1):
    return pl.pallas_call(
        _tcA_body,
        grid=(GRID,),
        in_specs=[
            pl.BlockSpec((2, BLK, 16), lambda i: (0, i, 0)),
            pl.BlockSpec((BLK, 128), lambda i: (i, 0)),
            pl.BlockSpec((128, 128), lambda i: (0, 0)),
        ],
        out_specs=[
            pl.BlockSpec((BLK, 1), lambda i: (i, 0)),
            pl.BlockSpec((BLK, 64), lambda i: (i, 0)),
            pl.BlockSpec((BLK, 64), lambda i: (i, 0)),
        ],
        out_shape=[
            jax.ShapeDtypeStruct((NP, 1), jnp.float32),
            jax.ShapeDtypeStruct((NP, 64), jnp.float32),
            jax.ShapeDtypeStruct((NP, 64), jnp.float32),
        ],
    )(deg2, xp, W1)


def _tcB(p, ga, gb, dinv, W2, b1):
    return pl.pallas_call(
        _tcB_body,
        grid=(GRID,),
        in_specs=[
            pl.BlockSpec((2, 2, BLK, 64), lambda i: (0, 0, i, 0)),
            pl.BlockSpec((BLK, 64), lambda i: (i, 0)),
            pl.BlockSpec((BLK, 64), lambda i: (i, 0)),
            pl.BlockSpec((BLK, 1), lambda i: (i, 0)),
            pl.BlockSpec((128, 16), lambda i: (0, 0)),
            pl.BlockSpec((1, 128), lambda i: (0, 0)),
        ],
        out_specs=pl.BlockSpec((BLK, 16), lambda i: (i, 0)),
        out_shape=jax.ShapeDtypeStruct((NP, 16), jnp.float32),
    )(p, ga, gb, dinv, W2, b1)


def _tcC(q, g2, dinv, b2):
    return pl.pallas_call(
        _tcC_body,
        grid=(GRID,),
        in_specs=[
            pl.BlockSpec((1, 2, BLK, 16), lambda i: (0, 0, i, 0)),
            pl.BlockSpec((BLK, 16), lambda i: (i, 0)),
            pl.BlockSpec((BLK, 1), lambda i: (i, 0)),
            pl.BlockSpec((1, 16), lambda i: (0, 0)),
        ],
        out_specs=pl.BlockSpec((BLK, 16), lambda i: (i, 0)),
        out_shape=jax.ShapeDtypeStruct((NP, 16), jnp.float32),
    )(q, g2, dinv, b2)


def kernel(x, edge_index, W1, b1, W2, b2):
    src = edge_index[0].astype(jnp.int32)
    dst = edge_index[1].astype(jnp.int32)
    pad = EP - E
    srcp = jnp.concatenate([src, jnp.full((pad,), N, jnp.int32)]).reshape(
        NW, CPW, CHUNK)
    dstp = jnp.concatenate([dst, jnp.full((pad,), N + 1, jnp.int32)]).reshape(
        NW, CPW, CHUNK)
    xp = jnp.zeros((NP, 128), jnp.float32).at[:N].set(x)
    ones_tile = jnp.ones((CHUNK, 16), jnp.float32)
    z16 = jnp.zeros((NP, 16), jnp.float32)
    z64 = jnp.zeros((NP, 64), jnp.float32)

    deg2 = _deg16(ones_tile, dstp, z16)
    dinv, ga, gb = _tcA(deg2, xp, W1)
    p = _prop64x2([ga, gb], srcp, dstp, z64)
    g2 = _tcB(p, ga, gb, dinv, W2, b1.reshape(1, 128))
    q = _prop16x1([g2], srcp, dstp, z16)
    outp = _tcC(q, g2, dinv, b2.reshape(1, 16))
    return outp[:N]


# speedup vs baseline: 32.1161x; 1.0010x over previous
import functools

import jax
import jax.numpy as jnp
from jax import lax
from jax.experimental import pallas as pl
from jax.experimental.pallas import tpu as pltpu
from jax.experimental.pallas import tpu_sc as plsc

N = 10000
NP = 10240
E = 320000
NW = 32
CHUNK = 64
CPW = 160
EP = NW * CPW * CHUNK
ROWS_PER_TILE = NP // 16
RING = 4
BLK = 1024
GRID = NP // BLK


def _make_prop(TW, NPASS):
    mesh = plsc.VectorSubcoreMesh(core_axis_name="c", subcore_axis_name="s")

    @functools.partial(
        pl.kernel,
        out_type=jax.ShapeDtypeStruct((NPASS, 2, NP, TW), jnp.float32),
        mesh=mesh,
        compiler_params=pltpu.CompilerParams(use_tc_tiling_on_sc=False),
        scratch_types=[
            pltpu.VMEM((CPW, CHUNK), jnp.int32),
            pltpu.VMEM((CPW, CHUNK), jnp.int32),
            [pltpu.VMEM((CHUNK, TW), jnp.float32)] * RING,
            pltpu.VMEM_SHARED((NP, TW), jnp.float32),
            pltpu.VMEM_SHARED((NP, TW), jnp.float32),
            [pltpu.SemaphoreType.DMA] * RING,
            [pltpu.SemaphoreType.DMA] * RING,
        ],
    )
    def prop(gs, srcw, dstw, zinit, out,
             src_v, dst_v, rows_v, gtab, acc, gsem, ssem):
        c = lax.axis_index("c")
        s = lax.axis_index("s")
        wid = s * 2 + c
        r0 = s * ROWS_PER_TILE
        pltpu.sync_copy(srcw.at[wid], src_v)
        pltpu.sync_copy(dstw.at[wid], dst_v)

        for p in range(NPASS):
            pltpu.sync_copy(gs[p].at[pl.ds(r0, ROWS_PER_TILE)],
                            gtab.at[pl.ds(r0, ROWS_PER_TILE)])
            pltpu.sync_copy(zinit.at[pl.ds(r0, ROWS_PER_TILE)],
                            acc.at[pl.ds(r0, ROWS_PER_TILE)])
            plsc.subcore_barrier()

            for b in range(RING - 1):
                pltpu.async_copy(gtab.at[src_v.at[b]], rows_v[b], gsem[b])

            def body(jj, carry):
                for b in range(RING):
                    j = jj * RING + b
                    pltpu.make_async_copy(
                        gtab.at[src_v.at[j]], rows_v[b], gsem[b]).wait()
                    pltpu.async_copy(
                        rows_v[b], acc.at[dst_v.at[j]], ssem[b], add=True)

                    bn = (b + RING - 1) % RING

                    @pl.when(j + RING - 1 < CPW)
                    def _():
                        @pl.when(j >= 1)
                        def _():
                            pltpu.make_async_copy(
                                rows_v[bn], acc.at[dst_v.at[j]],
                                ssem[bn]).wait()
                        pltpu.async_copy(
                            gtab.at[src_v.at[j + RING - 1]], rows_v[bn],
                            gsem[bn])
                return carry

            lax.fori_loop(0, CPW // RING, body, 0)
            for b in range(RING):
                pltpu.make_async_copy(
                    rows_v[b], acc.at[dst_v.at[b]], ssem[b]).wait()
            plsc.subcore_barrier()
            pltpu.sync_copy(acc.at[pl.ds(r0, ROWS_PER_TILE)],
                            out.at[p, c, pl.ds(r0, ROWS_PER_TILE)])

    return prop


_prop64x2 = _make_prop(64, 2)
_prop16x1 = _make_prop(16, 1)


def _make_deg():
    mesh = plsc.VectorSubcoreMesh(core_axis_name="c", subcore_axis_name="s")

    @functools.partial(
        pl.kernel,
        out_type=jax.ShapeDtypeStruct((2, NP, 16), jnp.float32),
        mesh=mesh,
        compiler_params=pltpu.CompilerParams(use_tc_tiling_on_sc=False),
        scratch_types=[
            pltpu.VMEM((CPW, CHUNK), jnp.int32),
            pltpu.VMEM((CHUNK, 16), jnp.float32),
            pltpu.VMEM_SHARED((NP, 16), jnp.float32),
            pltpu.SemaphoreType.DMA,
        ],
    )
    def deg(ones_hbm, dstw, zinit, out, dst_v, ones_v, acc, sem):
        c = lax.axis_index("c")
        s = lax.axis_index("s")
        wid = s * 2 + c
        r0 = s * ROWS_PER_TILE
        pltpu.sync_copy(zinit.at[pl.ds(r0, ROWS_PER_TILE)],
                        acc.at[pl.ds(r0, ROWS_PER_TILE)])
        pltpu.sync_copy(dstw.at[wid], dst_v)
        pltpu.sync_copy(ones_hbm, ones_v)
        plsc.subcore_barrier()

        def fire(j, carry):
            pltpu.async_copy(ones_v, acc.at[dst_v.at[j]], sem, add=True)
            return carry

        lax.fori_loop(0, CPW, fire, 0)

        def drain(j, carry):
            pltpu.make_async_copy(ones_v, acc.at[dst_v.at[j]], sem).wait()
            return carry

        lax.fori_loop(0, CPW, drain, 0)
        plsc.subcore_barrier()
        pltpu.sync_copy(acc.at[pl.ds(r0, ROWS_PER_TILE)],
                        out.at[c, pl.ds(r0, ROWS_PER_TILE)])

    return deg


_deg16 = _make_deg()


def _tcA1_body(x_ref, w1_ref, h_ref):
    h_ref[...] = jnp.dot(x_ref[...], w1_ref[...],
                         preferred_element_type=jnp.float32)


def _tcA2_body(deg2_ref, h_ref, dinv_ref, ga_ref, gb_ref):
    deg = deg2_ref[0, :, 0] + deg2_ref[1, :, 0] + 1.0
    dinv = lax.rsqrt(deg)
    dinv_ref[...] = dinv[:, None]
    g1 = h_ref[...] * dinv[:, None]
    ga_ref[...] = g1[:, :64]
    gb_ref[...] = g1[:, 64:]


def _tcB_body(p_ref, ga_ref, gb_ref, dinv_ref, w2_ref, b1_ref, g2_ref):
    dinv = dinv_ref[...]
    tot = jnp.concatenate(
        [ga_ref[...] + p_ref[0, 0] + p_ref[0, 1],
         gb_ref[...] + p_ref[1, 0] + p_ref[1, 1]], axis=1)
    z = jnp.maximum(tot * dinv + b1_ref[...], 0.0)
    h2 = jnp.dot(z, w2_ref[...], preferred_element_type=jnp.float32)
    g2_ref[...] = h2 * dinv


def _tcC_body(q_ref, g2_ref, dinv_ref, b2_ref, out_ref):
    out_ref[...] = ((g2_ref[...] + q_ref[0, 0] + q_ref[0, 1])
                    * dinv_ref[...] + b2_ref[...])


def _tcA1(xp, W1):
    return pl.pallas_call(
        _tcA1_body,
        grid=(GRID,),
        in_specs=[
            pl.BlockSpec((BLK, 128), lambda i: (i, 0)),
            pl.BlockSpec((128, 128), lambda i: (0, 0)),
        ],
        out_specs=pl.BlockSpec((BLK, 128), lambda i: (i, 0)),
        out_shape=jax.ShapeDtypeStruct((NP, 128), jnp.float32),
    )(xp, W1)


def _tcA2(deg2, h):
    return pl.pallas_call(
        _tcA2_body,
        grid=(GRID,),
        in_specs=[
            pl.BlockSpec((2, BLK, 16), lambda i: (0, i, 0)),
            pl.BlockSpec((BLK, 128), lambda i: (i, 0)),
        ],
        out_specs=[
            pl.BlockSpec((BLK, 1), lambda i: (i, 0)),
            pl.BlockSpec((BLK, 64), lambda i: (i, 0)),
            pl.BlockSpec((BLK, 64), lambda i: (i, 0)),
        ],
        out_shape=[
            jax.ShapeDtypeStruct((NP, 1), jnp.float32),
            jax.ShapeDtypeStruct((NP, 64), jnp.float32),
            jax.ShapeDtypeStruct((NP, 64), jnp.float32),
        ],
    )(deg2, h)


def _tcB(p, ga, gb, dinv, W2, b1):
    return pl.pallas_call(
        _tcB_body,
        grid=(GRID,),
        in_specs=[
            pl.BlockSpec((2, 2, BLK, 64), lambda i: (0, 0, i, 0)),
            pl.BlockSpec((BLK, 64), lambda i: (i, 0)),
            pl.BlockSpec((BLK, 64), lambda i: (i, 0)),
            pl.BlockSpec((BLK, 1), lambda i: (i, 0)),
            pl.BlockSpec((128, 16), lambda i: (0, 0)),
            pl.BlockSpec((1, 128), lambda i: (0, 0)),
        ],
        out_specs=pl.BlockSpec((BLK, 16), lambda i: (i, 0)),
        out_shape=jax.ShapeDtypeStruct((NP, 16), jnp.float32),
    )(p, ga, gb, dinv, W2, b1)


def _tcC(q, g2, dinv, b2):
    return pl.pallas_call(
        _tcC_body,
        grid=(GRID,),
        in_specs=[
            pl.BlockSpec((1, 2, BLK, 16), lambda i: (0, 0, i, 0)),
            pl.BlockSpec((BLK, 16), lambda i: (i, 0)),
            pl.BlockSpec((BLK, 1), lambda i: (i, 0)),
            pl.BlockSpec((1, 16), lambda i: (0, 0)),
        ],
        out_specs=pl.BlockSpec((BLK, 16), lambda i: (i, 0)),
        out_shape=jax.ShapeDtypeStruct((NP, 16), jnp.float32),
    )(q, g2, dinv, b2)


def kernel(x, edge_index, W1, b1, W2, b2):
    src = edge_index[0].astype(jnp.int32)
    dst = edge_index[1].astype(jnp.int32)
    pad = EP - E
    srcp = jnp.concatenate([src, jnp.full((pad,), N, jnp.int32)]).reshape(
        NW, CPW, CHUNK)
    dstp = jnp.concatenate([dst, jnp.full((pad,), N + 1, jnp.int32)]).reshape(
        NW, CPW, CHUNK)
    xp = jnp.zeros((NP, 128), jnp.float32).at[:N].set(x)
    ones_tile = jnp.ones((CHUNK, 16), jnp.float32)
    z16 = jnp.zeros((NP, 16), jnp.float32)
    z64 = jnp.zeros((NP, 64), jnp.float32)

    deg2 = _deg16(ones_tile, dstp, z16)
    h1 = _tcA1(xp, W1)
    dinv, ga, gb = _tcA2(deg2, h1)
    p = _prop64x2([ga, gb], srcp, dstp, z64)
    g2 = _tcB(p, ga, gb, dinv, W2, b1.reshape(1, 128))
    q = _prop16x1([g2], srcp, dstp, z16)
    outp = _tcC(q, g2, dinv, b2.reshape(1, 16))
    return outp[:N]


# speedup vs baseline: 35.0969x; 1.0928x over previous
import functools

import jax
import jax.numpy as jnp
from jax import lax
from jax.experimental import pallas as pl
from jax.experimental.pallas import tpu as pltpu
from jax.experimental.pallas import tpu_sc as plsc

N = 10000
NP = 10240
E = 320000
NW = 32
CHUNK = 64
CPW = 160
EP = NW * CPW * CHUNK
ROWS_PER_TILE = NP // 16
RING = 4
BLK = 1024
GRID = NP // BLK


def _make_prop(TW, NPASS):
    mesh = plsc.VectorSubcoreMesh(core_axis_name="c", subcore_axis_name="s")
    D = TW * NPASS

    @functools.partial(
        pl.kernel,
        out_type=jax.ShapeDtypeStruct((2, NP, D), jnp.float32),
        mesh=mesh,
        compiler_params=pltpu.CompilerParams(use_tc_tiling_on_sc=False),
        scratch_types=[
            pltpu.VMEM((CPW, CHUNK), jnp.int32),
            pltpu.VMEM((CPW, CHUNK), jnp.int32),
            [pltpu.VMEM((CHUNK, TW), jnp.float32)] * RING,
            pltpu.VMEM_SHARED((NP, TW), jnp.float32),
            pltpu.VMEM_SHARED((NP, TW), jnp.float32),
            [pltpu.SemaphoreType.DMA] * RING,
            [pltpu.SemaphoreType.DMA] * RING,
        ],
    )
    def prop(g, srcw, dstw, zinit, out,
             src_v, dst_v, rows_v, gtab, acc, gsem, ssem):
        c = lax.axis_index("c")
        s = lax.axis_index("s")
        wid = s * 2 + c
        r0 = s * ROWS_PER_TILE
        pltpu.sync_copy(srcw.at[wid], src_v)
        pltpu.sync_copy(dstw.at[wid], dst_v)

        for p in range(NPASS):
            pltpu.sync_copy(g.at[pl.ds(r0, ROWS_PER_TILE), pl.ds(p * TW, TW)],
                            gtab.at[pl.ds(r0, ROWS_PER_TILE)])
            pltpu.sync_copy(zinit.at[pl.ds(r0, ROWS_PER_TILE)],
                            acc.at[pl.ds(r0, ROWS_PER_TILE)])
            plsc.subcore_barrier()

            for b in range(RING - 1):
                pltpu.async_copy(gtab.at[src_v.at[b]], rows_v[b], gsem[b])

            def body(jj, carry):
                for b in range(RING):
                    j = jj * RING + b
                    pltpu.make_async_copy(
                        gtab.at[src_v.at[j]], rows_v[b], gsem[b]).wait()
                    pltpu.async_copy(
                        rows_v[b], acc.at[dst_v.at[j]], ssem[b], add=True)

                    bn = (b + RING - 1) % RING

                    @pl.when(j + RING - 1 < CPW)
                    def _():
                        @pl.when(j >= 1)
                        def _():
                            pltpu.make_async_copy(
                                rows_v[bn], acc.at[dst_v.at[j]],
                                ssem[bn]).wait()
                        pltpu.async_copy(
                            gtab.at[src_v.at[j + RING - 1]], rows_v[bn],
                            gsem[bn])
                return carry

            lax.fori_loop(0, CPW // RING, body, 0)
            for b in range(RING):
                pltpu.make_async_copy(
                    rows_v[b], acc.at[dst_v.at[b]], ssem[b]).wait()
            plsc.subcore_barrier()
            pltpu.sync_copy(
                acc.at[pl.ds(r0, ROWS_PER_TILE)],
                out.at[c, pl.ds(r0, ROWS_PER_TILE), pl.ds(p * TW, TW)])

    return prop


_prop64x2 = _make_prop(64, 2)
_prop16x1 = _make_prop(16, 1)


def _make_deg():
    mesh = plsc.VectorSubcoreMesh(core_axis_name="c", subcore_axis_name="s")

    @functools.partial(
        pl.kernel,
        out_type=jax.ShapeDtypeStruct((2, NP, 8), jnp.float32),
        mesh=mesh,
        compiler_params=pltpu.CompilerParams(use_tc_tiling_on_sc=False),
        scratch_types=[
            pltpu.VMEM((CPW, CHUNK), jnp.int32),
            pltpu.VMEM((CHUNK, 16), jnp.float32),
            pltpu.VMEM_SHARED((NP, 16), jnp.float32),
            pltpu.SemaphoreType.DMA,
        ],
    )
    def deg(ones_hbm, dstw, zinit, out, dst_v, ones_v, acc, sem):
        c = lax.axis_index("c")
        s = lax.axis_index("s")
        wid = s * 2 + c
        r0 = s * ROWS_PER_TILE
        pltpu.sync_copy(zinit.at[pl.ds(r0, ROWS_PER_TILE)],
                        acc.at[pl.ds(r0, ROWS_PER_TILE)])
        pltpu.sync_copy(dstw.at[wid], dst_v)
        pltpu.sync_copy(ones_hbm, ones_v)
        plsc.subcore_barrier()

        def fire(j, carry):
            pltpu.async_copy(ones_v, acc.at[dst_v.at[j]], sem, add=True)
            return carry

        lax.fori_loop(0, CPW, fire, 0)

        def drain(j, carry):
            pltpu.make_async_copy(ones_v, acc.at[dst_v.at[j]], sem).wait()
            return carry

        lax.fori_loop(0, CPW, drain, 0)
        plsc.subcore_barrier()
        pltpu.sync_copy(acc.at[pl.ds(r0, ROWS_PER_TILE), pl.ds(0, 8)],
                        out.at[c, pl.ds(r0, ROWS_PER_TILE)])

    return deg


_deg16 = _make_deg()


def _tcA1_body(x_ref, w1_ref, h_ref):
    h_ref[...] = jnp.dot(x_ref[...], w1_ref[...],
                         preferred_element_type=jnp.float32)


def _tcA2_body(deg2_ref, h_ref, dinv_ref, g1_ref):
    deg = deg2_ref[0, :, 0] + deg2_ref[1, :, 0] + 1.0
    dinv = lax.rsqrt(deg)
    dinv_ref[...] = dinv[:, None]
    g1_ref[...] = h_ref[...] * dinv[:, None]


def _tcB_body(p_ref, g1_ref, dinv_ref, w2_ref, b1_ref, g2_ref):
    dinv = dinv_ref[...]
    tot = g1_ref[...] + p_ref[0] + p_ref[1]
    z = jnp.maximum(tot * dinv + b1_ref[...], 0.0)
    h2 = jnp.dot(z, w2_ref[...], preferred_element_type=jnp.float32)
    g2_ref[...] = h2 * dinv


def _tcC_body(q_ref, g2_ref, dinv_ref, b2_ref, out_ref):
    out_ref[...] = ((g2_ref[...] + q_ref[0] + q_ref[1])
                    * dinv_ref[...] + b2_ref[...])


def _tcA1(xp, W1):
    return pl.pallas_call(
        _tcA1_body,
        grid=(GRID,),
        in_specs=[
            pl.BlockSpec((BLK, 128), lambda i: (i, 0)),
            pl.BlockSpec((128, 128), lambda i: (0, 0)),
        ],
        out_specs=pl.BlockSpec((BLK, 128), lambda i: (i, 0)),
        out_shape=jax.ShapeDtypeStruct((NP, 128), jnp.float32),
    )(xp, W1)


def _tcA2(deg2, h):
    return pl.pallas_call(
        _tcA2_body,
        grid=(GRID,),
        in_specs=[
            pl.BlockSpec((2, BLK, 8), lambda i: (0, i, 0)),
            pl.BlockSpec((BLK, 128), lambda i: (i, 0)),
        ],
        out_specs=[
            pl.BlockSpec((BLK, 1), lambda i: (i, 0)),
            pl.BlockSpec((BLK, 128), lambda i: (i, 0)),
        ],
        out_shape=[
            jax.ShapeDtypeStruct((NP, 1), jnp.float32),
            jax.ShapeDtypeStruct((NP, 128), jnp.float32),
        ],
    )(deg2, h)


def _tcB(p, g1, dinv, W2, b1):
    return pl.pallas_call(
        _tcB_body,
        grid=(GRID,),
        in_specs=[
            pl.BlockSpec((2, BLK, 128), lambda i: (0, i, 0)),
            pl.BlockSpec((BLK, 128), lambda i: (i, 0)),
            pl.BlockSpec((BLK, 1), lambda i: (i, 0)),
            pl.BlockSpec((128, 16), lambda i: (0, 0)),
            pl.BlockSpec((1, 128), lambda i: (0, 0)),
        ],
        out_specs=pl.BlockSpec((BLK, 16), lambda i: (i, 0)),
        out_shape=jax.ShapeDtypeStruct((NP, 16), jnp.float32),
    )(p, g1, dinv, W2, b1)


def _tcC(q, g2, dinv, b2):
    return pl.pallas_call(
        _tcC_body,
        grid=(GRID,),
        in_specs=[
            pl.BlockSpec((2, BLK, 16), lambda i: (0, i, 0)),
            pl.BlockSpec((BLK, 16), lambda i: (i, 0)),
            pl.BlockSpec((BLK, 1), lambda i: (i, 0)),
            pl.BlockSpec((1, 16), lambda i: (0, 0)),
        ],
        out_specs=pl.BlockSpec((BLK, 16), lambda i: (i, 0)),
        out_shape=jax.ShapeDtypeStruct((NP, 16), jnp.float32),
    )(q, g2, dinv, b2)


def kernel(x, edge_index, W1, b1, W2, b2):
    src = edge_index[0].astype(jnp.int32)
    dst = edge_index[1].astype(jnp.int32)
    pad = EP - E
    srcp = jnp.concatenate([src, jnp.full((pad,), N, jnp.int32)]).reshape(
        NW, CPW, CHUNK)
    dstp = jnp.concatenate([dst, jnp.full((pad,), N + 1, jnp.int32)]).reshape(
        NW, CPW, CHUNK)
    xp = jnp.zeros((NP, 128), jnp.float32).at[:N].set(x)
    ones_tile = jnp.ones((CHUNK, 16), jnp.float32)
    z16 = jnp.zeros((NP, 16), jnp.float32)
    z64 = jnp.zeros((NP, 64), jnp.float32)

    deg2 = _deg16(ones_tile, dstp, z16)
    h1 = _tcA1(xp, W1)
    dinv, g1 = _tcA2(deg2, h1)
    p = _prop64x2(g1, srcp, dstp, z64)
    g2 = _tcB(p, g1, dinv, W2, b1.reshape(1, 128))
    q = _prop16x1(g2, srcp, dstp, z16)
    outp = _tcC(q, g2, dinv, b2.reshape(1, 16))
    return outp[:N]


# speedup vs baseline: 35.5321x; 1.0124x over previous
import functools

import jax
import jax.numpy as jnp
from jax import lax
from jax.experimental import pallas as pl
from jax.experimental.pallas import tpu as pltpu
from jax.experimental.pallas import tpu_sc as plsc

N = 10000
NP = 10240
E = 320000
NW = 32
CHUNK = 64
CPW = 160
EP = NW * CPW * CHUNK
ROWS_PER_TILE = NP // 16
ROWS_STAGE = N // 16
RING = 4
BLK = 1000
GRID = N // BLK


def _make_prop(TW, NPASS):
    mesh = plsc.VectorSubcoreMesh(core_axis_name="c", subcore_axis_name="s")
    D = TW * NPASS

    @functools.partial(
        pl.kernel,
        out_type=jax.ShapeDtypeStruct((2, NP, D), jnp.float32),
        mesh=mesh,
        compiler_params=pltpu.CompilerParams(use_tc_tiling_on_sc=False),
        scratch_types=[
            pltpu.VMEM((CPW, CHUNK), jnp.int32),
            pltpu.VMEM((CPW, CHUNK), jnp.int32),
            [pltpu.VMEM((CHUNK, TW), jnp.float32)] * RING,
            pltpu.VMEM_SHARED((NP, TW), jnp.float32),
            pltpu.VMEM_SHARED((NP, TW), jnp.float32),
            [pltpu.SemaphoreType.DMA] * RING,
            [pltpu.SemaphoreType.DMA] * RING,
        ],
    )
    def prop(g, srcw, dstw, zinit, out,
             src_v, dst_v, rows_v, gtab, acc, gsem, ssem):
        c = lax.axis_index("c")
        s = lax.axis_index("s")
        wid = s * 2 + c
        r0 = s * ROWS_PER_TILE
        pltpu.sync_copy(srcw.at[wid], src_v)
        pltpu.sync_copy(dstw.at[wid], dst_v)

        rs = s * ROWS_STAGE

        for p in range(NPASS):
            pltpu.sync_copy(g.at[pl.ds(rs, ROWS_STAGE), pl.ds(p * TW, TW)],
                            gtab.at[pl.ds(rs, ROWS_STAGE)])
            pltpu.sync_copy(zinit.at[pl.ds(r0, ROWS_PER_TILE)],
                            acc.at[pl.ds(r0, ROWS_PER_TILE)])
            plsc.subcore_barrier()

            for b in range(RING - 1):
                pltpu.async_copy(gtab.at[src_v.at[b]], rows_v[b], gsem[b])

            def body(jj, carry):
                for b in range(RING):
                    j = jj * RING + b
                    pltpu.make_async_copy(
                        gtab.at[src_v.at[j]], rows_v[b], gsem[b]).wait()
                    pltpu.async_copy(
                        rows_v[b], acc.at[dst_v.at[j]], ssem[b], add=True)

                    bn = (b + RING - 1) % RING

                    @pl.when(j + RING - 1 < CPW)
                    def _():
                        @pl.when(j >= 1)
                        def _():
                            pltpu.make_async_copy(
                                rows_v[bn], acc.at[dst_v.at[j]],
                                ssem[bn]).wait()
                        pltpu.async_copy(
                            gtab.at[src_v.at[j + RING - 1]], rows_v[bn],
                            gsem[bn])
                return carry

            lax.fori_loop(0, CPW // RING, body, 0)
            for b in range(RING):
                pltpu.make_async_copy(
                    rows_v[b], acc.at[dst_v.at[b]], ssem[b]).wait()
            plsc.subcore_barrier()
            pltpu.sync_copy(
                acc.at[pl.ds(r0, ROWS_PER_TILE)],
                out.at[c, pl.ds(r0, ROWS_PER_TILE), pl.ds(p * TW, TW)])

    return prop


_prop64x2 = _make_prop(64, 2)
_prop16x1 = _make_prop(16, 1)


def _make_deg():
    mesh = plsc.VectorSubcoreMesh(core_axis_name="c", subcore_axis_name="s")

    @functools.partial(
        pl.kernel,
        out_type=jax.ShapeDtypeStruct((2, NP, 8), jnp.float32),
        mesh=mesh,
        compiler_params=pltpu.CompilerParams(use_tc_tiling_on_sc=False),
        scratch_types=[
            pltpu.VMEM((CPW, CHUNK), jnp.int32),
            pltpu.VMEM((CHUNK, 16), jnp.float32),
            pltpu.VMEM_SHARED((NP, 16), jnp.float32),
            pltpu.SemaphoreType.DMA,
        ],
    )
    def deg(ones_hbm, dstw, zinit, out, dst_v, ones_v, acc, sem):
        c = lax.axis_index("c")
        s = lax.axis_index("s")
        wid = s * 2 + c
        r0 = s * ROWS_PER_TILE
        pltpu.sync_copy(zinit.at[pl.ds(r0, ROWS_PER_TILE)],
                        acc.at[pl.ds(r0, ROWS_PER_TILE)])
        pltpu.sync_copy(dstw.at[wid], dst_v)
        pltpu.sync_copy(ones_hbm, ones_v)
        plsc.subcore_barrier()

        def fire(j, carry):
            pltpu.async_copy(ones_v, acc.at[dst_v.at[j]], sem, add=True)
            return carry

        lax.fori_loop(0, CPW, fire, 0)

        def drain(j, carry):
            pltpu.make_async_copy(ones_v, acc.at[dst_v.at[j]], sem).wait()
            return carry

        lax.fori_loop(0, CPW, drain, 0)
        plsc.subcore_barrier()
        pltpu.sync_copy(acc.at[pl.ds(r0, ROWS_PER_TILE), pl.ds(0, 8)],
                        out.at[c, pl.ds(r0, ROWS_PER_TILE)])

    return deg


_deg16 = _make_deg()


def _tcA1_body(x_ref, w1_ref, h_ref):
    h_ref[...] = jnp.dot(x_ref[...], w1_ref[...],
                         preferred_element_type=jnp.float32)


def _tcA2_body(deg2_ref, h_ref, dinv_ref, g1_ref):
    deg = deg2_ref[0, :, 0] + deg2_ref[1, :, 0] + 1.0
    dinv = lax.rsqrt(deg)
    dinv_ref[...] = dinv[:, None]
    g1_ref[...] = h_ref[...] * dinv[:, None]


def _tcB_body(p_ref, g1_ref, dinv_ref, w2_ref, b1_ref, g2_ref):
    dinv = dinv_ref[...]
    tot = g1_ref[...] + p_ref[0] + p_ref[1]
    z = jnp.maximum(tot * dinv + b1_ref[...], 0.0)
    h2 = jnp.dot(z, w2_ref[...], preferred_element_type=jnp.float32)
    g2_ref[...] = h2 * dinv


def _tcC_body(q_ref, g2_ref, dinv_ref, b2_ref, out_ref):
    out_ref[...] = ((g2_ref[...] + q_ref[0] + q_ref[1])
                    * dinv_ref[...] + b2_ref[...])


def _tcA1(xp, W1):
    return pl.pallas_call(
        _tcA1_body,
        grid=(GRID,),
        in_specs=[
            pl.BlockSpec((BLK, 128), lambda i: (i, 0)),
            pl.BlockSpec((128, 128), lambda i: (0, 0)),
        ],
        out_specs=pl.BlockSpec((BLK, 128), lambda i: (i, 0)),
        out_shape=jax.ShapeDtypeStruct((N, 128), jnp.float32),
    )(xp, W1)


def _tcA2(deg2, h):
    return pl.pallas_call(
        _tcA2_body,
        grid=(GRID,),
        in_specs=[
            pl.BlockSpec((2, BLK, 8), lambda i: (0, i, 0)),
            pl.BlockSpec((BLK, 128), lambda i: (i, 0)),
        ],
        out_specs=[
            pl.BlockSpec((BLK, 1), lambda i: (i, 0)),
            pl.BlockSpec((BLK, 128), lambda i: (i, 0)),
        ],
        out_shape=[
            jax.ShapeDtypeStruct((N, 1), jnp.float32),
            jax.ShapeDtypeStruct((N, 128), jnp.float32),
        ],
    )(deg2, h)


def _tcB(p, g1, dinv, W2, b1):
    return pl.pallas_call(
        _tcB_body,
        grid=(GRID,),
        in_specs=[
            pl.BlockSpec((2, BLK, 128), lambda i: (0, i, 0)),
            pl.BlockSpec((BLK, 128), lambda i: (i, 0)),
            pl.BlockSpec((BLK, 1), lambda i: (i, 0)),
            pl.BlockSpec((128, 16), lambda i: (0, 0)),
            pl.BlockSpec((1, 128), lambda i: (0, 0)),
        ],
        out_specs=pl.BlockSpec((BLK, 16), lambda i: (i, 0)),
        out_shape=jax.ShapeDtypeStruct((N, 16), jnp.float32),
    )(p, g1, dinv, W2, b1)


def _tcC(q, g2, dinv, b2):
    return pl.pallas_call(
        _tcC_body,
        grid=(GRID,),
        in_specs=[
            pl.BlockSpec((2, BLK, 16), lambda i: (0, i, 0)),
            pl.BlockSpec((BLK, 16), lambda i: (i, 0)),
            pl.BlockSpec((BLK, 1), lambda i: (i, 0)),
            pl.BlockSpec((1, 16), lambda i: (0, 0)),
        ],
        out_specs=pl.BlockSpec((BLK, 16), lambda i: (i, 0)),
        out_shape=jax.ShapeDtypeStruct((N, 16), jnp.float32),
    )(q, g2, dinv, b2)


def kernel(x, edge_index, W1, b1, W2, b2):
    src = edge_index[0].astype(jnp.int32)
    dst = edge_index[1].astype(jnp.int32)
    pad = EP - E
    srcp = jnp.concatenate([src, jnp.full((pad,), N, jnp.int32)]).reshape(
        NW, CPW, CHUNK)
    dstp = jnp.concatenate([dst, jnp.full((pad,), N + 1, jnp.int32)]).reshape(
        NW, CPW, CHUNK)
    ones_tile = jnp.ones((CHUNK, 16), jnp.float32)
    z16 = jnp.zeros((NP, 16), jnp.float32)
    z64 = jnp.zeros((NP, 64), jnp.float32)

    deg2 = _deg16(ones_tile, dstp, z16)
    h1 = _tcA1(x, W1)
    dinv, g1 = _tcA2(deg2, h1)
    p = _prop64x2(g1, srcp, dstp, z64)
    g2 = _tcB(p, g1, dinv, W2, b1.reshape(1, 128))
    q = _prop16x1(g2, srcp, dstp, z16)
    return _tcC(q, g2, dinv, b2.reshape(1, 16))


# speedup vs baseline: 36.3281x; 1.0224x over previous
import functools

import jax
import jax.numpy as jnp
from jax import lax
from jax.experimental import pallas as pl
from jax.experimental.pallas import tpu as pltpu
from jax.experimental.pallas import tpu_sc as plsc

N = 10000
NP = 10240
E = 320000
NW = 32
CHUNK = 64
CPW = 160
EP = NW * CPW * CHUNK
ROWS_PER_TILE = NP // 16
ROWS_STAGE = N // 16
RING = 4
BLK = 1000
GRID = N // BLK


def _make_prop(TW, NPASS):
    mesh = plsc.VectorSubcoreMesh(core_axis_name="c", subcore_axis_name="s")
    D = TW * NPASS

    @functools.partial(
        pl.kernel,
        out_type=jax.ShapeDtypeStruct((2, NP, D), jnp.float32),
        mesh=mesh,
        compiler_params=pltpu.CompilerParams(use_tc_tiling_on_sc=False),
        scratch_types=[
            pltpu.VMEM((CPW, CHUNK), jnp.int32),
            pltpu.VMEM((CPW, CHUNK), jnp.int32),
            [pltpu.VMEM((CHUNK, TW), jnp.float32)] * RING,
            pltpu.VMEM_SHARED((NP, TW), jnp.float32),
            pltpu.VMEM_SHARED((NP, TW), jnp.float32),
            [pltpu.SemaphoreType.DMA] * RING,
            [pltpu.SemaphoreType.DMA] * RING,
        ],
    )
    def prop(g, srcw, dstw, zinit, out,
             src_v, dst_v, rows_v, gtab, acc, gsem, ssem):
        c = lax.axis_index("c")
        s = lax.axis_index("s")
        wid = s * 2 + c
        r0 = s * ROWS_PER_TILE
        pltpu.sync_copy(srcw.at[wid], src_v)
        pltpu.sync_copy(dstw.at[wid], dst_v)

        rs = s * ROWS_STAGE

        for p in range(NPASS):
            pltpu.sync_copy(g.at[pl.ds(rs, ROWS_STAGE), pl.ds(p * TW, TW)],
                            gtab.at[pl.ds(rs, ROWS_STAGE)])
            pltpu.sync_copy(zinit.at[pl.ds(r0, ROWS_PER_TILE)],
                            acc.at[pl.ds(r0, ROWS_PER_TILE)])
            plsc.subcore_barrier()

            for b in range(RING - 1):
                pltpu.async_copy(gtab.at[src_v.at[b]], rows_v[b], gsem[b])

            def body(jj, carry):
                for b in range(RING):
                    j = jj * RING + b
                    pltpu.make_async_copy(
                        gtab.at[src_v.at[j]], rows_v[b], gsem[b]).wait()
                    pltpu.async_copy(
                        rows_v[b], acc.at[dst_v.at[j]], ssem[b], add=True)

                    bn = (b + RING - 1) % RING

                    @pl.when(j + RING - 1 < CPW)
                    def _():
                        @pl.when(j >= 1)
                        def _():
                            pltpu.make_async_copy(
                                rows_v[bn], acc.at[dst_v.at[j]],
                                ssem[bn]).wait()
                        pltpu.async_copy(
                            gtab.at[src_v.at[j + RING - 1]], rows_v[bn],
                            gsem[bn])
                return carry

            lax.fori_loop(0, CPW // RING, body, 0)
            for b in range(RING):
                pltpu.make_async_copy(
                    rows_v[b], acc.at[dst_v.at[b]], ssem[b]).wait()
            plsc.subcore_barrier()
            pltpu.sync_copy(
                acc.at[pl.ds(r0, ROWS_PER_TILE)],
                out.at[c, pl.ds(r0, ROWS_PER_TILE), pl.ds(p * TW, TW)])

    return prop


_prop64x2 = _make_prop(64, 2)
_prop16x1 = _make_prop(16, 1)


def _make_deg():
    mesh = plsc.VectorSubcoreMesh(core_axis_name="c", subcore_axis_name="s")

    @functools.partial(
        pl.kernel,
        out_type=jax.ShapeDtypeStruct((2, NP, 8), jnp.float32),
        mesh=mesh,
        compiler_params=pltpu.CompilerParams(use_tc_tiling_on_sc=False),
        scratch_types=[
            pltpu.VMEM((CPW, CHUNK), jnp.int32),
            pltpu.VMEM((CHUNK, 16), jnp.float32),
            pltpu.VMEM_SHARED((NP, 16), jnp.float32),
            pltpu.SemaphoreType.DMA,
        ],
    )
    def deg(ones_hbm, dstw, zinit, out, dst_v, ones_v, acc, sem):
        c = lax.axis_index("c")
        s = lax.axis_index("s")
        wid = s * 2 + c
        r0 = s * ROWS_PER_TILE
        pltpu.sync_copy(zinit.at[pl.ds(r0, ROWS_PER_TILE)],
                        acc.at[pl.ds(r0, ROWS_PER_TILE)])
        pltpu.sync_copy(dstw.at[wid], dst_v)
        pltpu.sync_copy(ones_hbm, ones_v)
        plsc.subcore_barrier()

        def fire(j, carry):
            pltpu.async_copy(ones_v, acc.at[dst_v.at[j]], sem, add=True)
            return carry

        lax.fori_loop(0, CPW, fire, 0)

        def drain(j, carry):
            pltpu.make_async_copy(ones_v, acc.at[dst_v.at[j]], sem).wait()
            return carry

        lax.fori_loop(0, CPW, drain, 0)
        plsc.subcore_barrier()
        pltpu.sync_copy(acc.at[pl.ds(r0, ROWS_PER_TILE), pl.ds(0, 8)],
                        out.at[c, pl.ds(r0, ROWS_PER_TILE)])

    return deg


_deg16 = _make_deg()


def _tcA1_body(x_ref, w1_ref, h_ref):
    h_ref[...] = jnp.dot(x_ref[...], w1_ref[...],
                         preferred_element_type=jnp.float32)


def _tcA2_body(deg2_ref, h_ref, dinv_ref, g1_ref):
    deg = deg2_ref[0, :, 0] + deg2_ref[1, :, 0] + 1.0
    dinv = lax.rsqrt(deg)
    dinv_ref[...] = dinv[:, None]
    g1_ref[...] = h_ref[...] * dinv[:, None]


def _tcB_body(p_ref, g1_ref, dinv_ref, w2_ref, b1_ref, g2_ref):
    dinv = dinv_ref[...]
    tot = g1_ref[...] + p_ref[0] + p_ref[1]
    z = jnp.maximum(tot * dinv + b1_ref[...], 0.0)
    h2 = jnp.dot(z, w2_ref[...], preferred_element_type=jnp.float32)
    g2_ref[...] = h2 * dinv


def _tcC_body(q_ref, g2_ref, dinv_ref, b2_ref, out_ref):
    out_ref[...] = ((g2_ref[...] + q_ref[0] + q_ref[1])
                    * dinv_ref[...] + b2_ref[...])


def _tcA1(xp, W1):
    return pl.pallas_call(
        _tcA1_body,
        grid=(GRID,),
        in_specs=[
            pl.BlockSpec((BLK, 128), lambda i: (i, 0)),
            pl.BlockSpec((128, 128), lambda i: (0, 0)),
        ],
        out_specs=pl.BlockSpec((BLK, 128), lambda i: (i, 0)),
        out_shape=jax.ShapeDtypeStruct((N, 128), jnp.float32),
    )(xp, W1)


def _tcA2(deg2, h):
    return pl.pallas_call(
        _tcA2_body,
        grid=(GRID,),
        in_specs=[
            pl.BlockSpec((2, BLK, 8), lambda i: (0, i, 0)),
            pl.BlockSpec((BLK, 128), lambda i: (i, 0)),
        ],
        out_specs=[
            pl.BlockSpec((BLK, 1), lambda i: (i, 0)),
            pl.BlockSpec((BLK, 128), lambda i: (i, 0)),
        ],
        out_shape=[
            jax.ShapeDtypeStruct((N, 1), jnp.float32),
            jax.ShapeDtypeStruct((N, 128), jnp.float32),
        ],
    )(deg2, h)


def _tcB(p, g1, dinv, W2, b1):
    return pl.pallas_call(
        _tcB_body,
        grid=(GRID,),
        in_specs=[
            pl.BlockSpec((2, BLK, 128), lambda i: (0, i, 0)),
            pl.BlockSpec((BLK, 128), lambda i: (i, 0)),
            pl.BlockSpec((BLK, 1), lambda i: (i, 0)),
            pl.BlockSpec((128, 16), lambda i: (0, 0)),
            pl.BlockSpec((1, 128), lambda i: (0, 0)),
        ],
        out_specs=pl.BlockSpec((BLK, 16), lambda i: (i, 0)),
        out_shape=jax.ShapeDtypeStruct((N, 16), jnp.float32),
    )(p, g1, dinv, W2, b1)


def _tcC(q, g2, dinv, b2):
    return pl.pallas_call(
        _tcC_body,
        grid=(GRID,),
        in_specs=[
            pl.BlockSpec((2, BLK, 16), lambda i: (0, i, 0)),
            pl.BlockSpec((BLK, 16), lambda i: (i, 0)),
            pl.BlockSpec((BLK, 1), lambda i: (i, 0)),
            pl.BlockSpec((1, 16), lambda i: (0, 0)),
        ],
        out_specs=pl.BlockSpec((BLK, 16), lambda i: (i, 0)),
        out_shape=jax.ShapeDtypeStruct((N, 16), jnp.float32),
    )(q, g2, dinv, b2)


def kernel(x, edge_index, W1, b1, W2, b2):
    pad = EP - E
    padv = jnp.array([[N], [N + 1]], jnp.int32) * jnp.ones(
        (2, pad), jnp.int32)
    ep = jnp.concatenate([edge_index.astype(jnp.int32), padv], axis=1)
    srcp = ep[0].reshape(NW, CPW, CHUNK)
    dstp = ep[1].reshape(NW, CPW, CHUNK)
    ones_tile = jnp.ones((CHUNK, 16), jnp.float32)
    z16 = jnp.zeros((NP, 16), jnp.float32)
    z64 = jnp.zeros((NP, 64), jnp.float32)

    deg2 = _deg16(ones_tile, dstp, z16)
    h1 = _tcA1(x, W1)
    dinv, g1 = _tcA2(deg2, h1)
    p = _prop64x2(g1, srcp, dstp, z64)
    g2 = _tcB(p, g1, dinv, W2, b1.reshape(1, 128))
    q = _prop16x1(g2, srcp, dstp, z16)
    return _tcC(q, g2, dinv, b2.reshape(1, 16))


# speedup vs baseline: 36.3396x; 1.0003x over previous
import functools

import jax
import jax.numpy as jnp
from jax import lax
from jax.experimental import pallas as pl
from jax.experimental.pallas import tpu as pltpu
from jax.experimental.pallas import tpu_sc as plsc

N = 10000
NP = 10240
E = 320000
NW = 32
CHUNK = 64
CPW = 160
EP = NW * CPW * CHUNK
ROWS_PER_TILE = NP // 16
ROWS_STAGE = N // 16
RING = 4
BLK = 1000
GRID = N // BLK


def _make_prop(TW, NPASS, CH=CHUNK, CW=CPW):
    mesh = plsc.VectorSubcoreMesh(core_axis_name="c", subcore_axis_name="s")
    D = TW * NPASS

    @functools.partial(
        pl.kernel,
        out_type=jax.ShapeDtypeStruct((2, NP, D), jnp.float32),
        mesh=mesh,
        compiler_params=pltpu.CompilerParams(use_tc_tiling_on_sc=False),
        scratch_types=[
            pltpu.VMEM((CW, CH), jnp.int32),
            pltpu.VMEM((CW, CH), jnp.int32),
            [pltpu.VMEM((CH, TW), jnp.float32)] * RING,
            pltpu.VMEM_SHARED((NP, TW), jnp.float32),
            pltpu.VMEM_SHARED((NP, TW), jnp.float32),
            [pltpu.SemaphoreType.DMA] * RING,
            [pltpu.SemaphoreType.DMA] * RING,
        ],
    )
    def prop(g, srcw, dstw, zinit, out,
             src_v, dst_v, rows_v, gtab, acc, gsem, ssem):
        c = lax.axis_index("c")
        s = lax.axis_index("s")
        wid = s * 2 + c
        r0 = s * ROWS_PER_TILE
        pltpu.sync_copy(srcw.at[wid], src_v)
        pltpu.sync_copy(dstw.at[wid], dst_v)

        rs = s * ROWS_STAGE

        for p in range(NPASS):
            pltpu.sync_copy(g.at[pl.ds(rs, ROWS_STAGE), pl.ds(p * TW, TW)],
                            gtab.at[pl.ds(rs, ROWS_STAGE)])
            pltpu.sync_copy(zinit.at[pl.ds(r0, ROWS_PER_TILE)],
                            acc.at[pl.ds(r0, ROWS_PER_TILE)])
            plsc.subcore_barrier()

            for b in range(RING - 1):
                pltpu.async_copy(gtab.at[src_v.at[b]], rows_v[b], gsem[b])

            def body(jj, carry):
                for b in range(RING):
                    j = jj * RING + b
                    pltpu.make_async_copy(
                        gtab.at[src_v.at[j]], rows_v[b], gsem[b]).wait()
                    pltpu.async_copy(
                        rows_v[b], acc.at[dst_v.at[j]], ssem[b], add=True)

                    bn = (b + RING - 1) % RING

                    @pl.when(j + RING - 1 < CW)
                    def _():
                        @pl.when(j >= 1)
                        def _():
                            pltpu.make_async_copy(
                                rows_v[bn], acc.at[dst_v.at[j]],
                                ssem[bn]).wait()
                        pltpu.async_copy(
                            gtab.at[src_v.at[j + RING - 1]], rows_v[bn],
                            gsem[bn])
                return carry

            lax.fori_loop(0, CW // RING, body, 0)
            for b in range(RING):
                pltpu.make_async_copy(
                    rows_v[b], acc.at[dst_v.at[b]], ssem[b]).wait()
            plsc.subcore_barrier()
            pltpu.sync_copy(
                acc.at[pl.ds(r0, ROWS_PER_TILE)],
                out.at[c, pl.ds(r0, ROWS_PER_TILE), pl.ds(p * TW, TW)])

    return prop


_prop64x2 = _make_prop(64, 2)
_prop16x1 = _make_prop(16, 1, CH=128, CW=80)


def _make_deg():
    mesh = plsc.VectorSubcoreMesh(core_axis_name="c", subcore_axis_name="s")

    @functools.partial(
        pl.kernel,
        out_type=jax.ShapeDtypeStruct((2, NP, 8), jnp.float32),
        mesh=mesh,
        compiler_params=pltpu.CompilerParams(use_tc_tiling_on_sc=False),
        scratch_types=[
            pltpu.VMEM((CPW, CHUNK), jnp.int32),
            pltpu.VMEM((CHUNK, 16), jnp.float32),
            pltpu.VMEM_SHARED((NP, 16), jnp.float32),
            pltpu.SemaphoreType.DMA,
        ],
    )
    def deg(ones_hbm, dstw, zinit, out, dst_v, ones_v, acc, sem):
        c = lax.axis_index("c")
        s = lax.axis_index("s")
        wid = s * 2 + c
        r0 = s * ROWS_PER_TILE
        pltpu.sync_copy(zinit.at[pl.ds(r0, ROWS_PER_TILE)],
                        acc.at[pl.ds(r0, ROWS_PER_TILE)])
        pltpu.sync_copy(dstw.at[wid], dst_v)
        pltpu.sync_copy(ones_hbm, ones_v)
        plsc.subcore_barrier()

        def fire(j, carry):
            pltpu.async_copy(ones_v, acc.at[dst_v.at[j]], sem, add=True)
            return carry

        lax.fori_loop(0, CPW, fire, 0)

        def drain(j, carry):
            pltpu.make_async_copy(ones_v, acc.at[dst_v.at[j]], sem).wait()
            return carry

        lax.fori_loop(0, CPW, drain, 0)
        plsc.subcore_barrier()
        pltpu.sync_copy(acc.at[pl.ds(r0, ROWS_PER_TILE), pl.ds(0, 8)],
                        out.at[c, pl.ds(r0, ROWS_PER_TILE)])

    return deg


_deg16 = _make_deg()


def _tcA1_body(x_ref, w1_ref, h_ref):
    h_ref[...] = jnp.dot(x_ref[...], w1_ref[...],
                         preferred_element_type=jnp.float32)


def _tcA2_body(deg2_ref, h_ref, dinv_ref, g1_ref):
    deg = deg2_ref[0, :, 0] + deg2_ref[1, :, 0] + 1.0
    dinv = lax.rsqrt(deg)
    dinv_ref[...] = dinv[:, None]
    g1_ref[...] = h_ref[...] * dinv[:, None]


def _tcB_body(p_ref, g1_ref, dinv_ref, w2_ref, b1_ref, g2_ref):
    dinv = dinv_ref[...]
    tot = g1_ref[...] + p_ref[0] + p_ref[1]
    z = jnp.maximum(tot * dinv + b1_ref[...], 0.0)
    h2 = jnp.dot(z, w2_ref[...], preferred_element_type=jnp.float32)
    g2_ref[...] = h2 * dinv


def _tcC_body(q_ref, g2_ref, dinv_ref, b2_ref, out_ref):
    out_ref[...] = ((g2_ref[...] + q_ref[0] + q_ref[1])
                    * dinv_ref[...] + b2_ref[...])


def _tcA1(xp, W1):
    return pl.pallas_call(
        _tcA1_body,
        grid=(GRID,),
        in_specs=[
            pl.BlockSpec((BLK, 128), lambda i: (i, 0)),
            pl.BlockSpec((128, 128), lambda i: (0, 0)),
        ],
        out_specs=pl.BlockSpec((BLK, 128), lambda i: (i, 0)),
        out_shape=jax.ShapeDtypeStruct((N, 128), jnp.float32),
    )(xp, W1)


def _tcA2(deg2, h):
    return pl.pallas_call(
        _tcA2_body,
        grid=(GRID,),
        in_specs=[
            pl.BlockSpec((2, BLK, 8), lambda i: (0, i, 0)),
            pl.BlockSpec((BLK, 128), lambda i: (i, 0)),
        ],
        out_specs=[
            pl.BlockSpec((BLK, 1), lambda i: (i, 0)),
            pl.BlockSpec((BLK, 128), lambda i: (i, 0)),
        ],
        out_shape=[
            jax.ShapeDtypeStruct((N, 1), jnp.float32),
            jax.ShapeDtypeStruct((N, 128), jnp.float32),
        ],
    )(deg2, h)


def _tcB(p, g1, dinv, W2, b1):
    return pl.pallas_call(
        _tcB_body,
        grid=(GRID,),
        in_specs=[
            pl.BlockSpec((2, BLK, 128), lambda i: (0, i, 0)),
            pl.BlockSpec((BLK, 128), lambda i: (i, 0)),
            pl.BlockSpec((BLK, 1), lambda i: (i, 0)),
            pl.BlockSpec((128, 16), lambda i: (0, 0)),
            pl.BlockSpec((1, 128), lambda i: (0, 0)),
        ],
        out_specs=pl.BlockSpec((BLK, 16), lambda i: (i, 0)),
        out_shape=jax.ShapeDtypeStruct((N, 16), jnp.float32),
    )(p, g1, dinv, W2, b1)


def _tcC(q, g2, dinv, b2):
    return pl.pallas_call(
        _tcC_body,
        grid=(GRID,),
        in_specs=[
            pl.BlockSpec((2, BLK, 16), lambda i: (0, i, 0)),
            pl.BlockSpec((BLK, 16), lambda i: (i, 0)),
            pl.BlockSpec((BLK, 1), lambda i: (i, 0)),
            pl.BlockSpec((1, 16), lambda i: (0, 0)),
        ],
        out_specs=pl.BlockSpec((BLK, 16), lambda i: (i, 0)),
        out_shape=jax.ShapeDtypeStruct((N, 16), jnp.float32),
    )(q, g2, dinv, b2)


def kernel(x, edge_index, W1, b1, W2, b2):
    pad = EP - E
    padv = jnp.array([[N], [N + 1]], jnp.int32) * jnp.ones(
        (2, pad), jnp.int32)
    ep = jnp.concatenate([edge_index.astype(jnp.int32), padv], axis=1)
    srcp = ep[0].reshape(NW, CPW, CHUNK)
    dstp = ep[1].reshape(NW, CPW, CHUNK)
    ones_tile = jnp.ones((CHUNK, 16), jnp.float32)
    z16 = jnp.zeros((NP, 16), jnp.float32)
    z64 = jnp.zeros((NP, 64), jnp.float32)

    deg2 = _deg16(ones_tile, dstp, z16)
    h1 = _tcA1(x, W1)
    dinv, g1 = _tcA2(deg2, h1)
    p = _prop64x2(g1, srcp, dstp, z64)
    g2 = _tcB(p, g1, dinv, W2, b1.reshape(1, 128))
    srcp2 = ep[0].reshape(NW, 80, 128)
    dstp2 = ep[1].reshape(NW, 80, 128)
    q = _prop16x1(g2, srcp2, dstp2, z16)
    return _tcC(q, g2, dinv, b2.reshape(1, 16))
